# Initial kernel scaffold; baseline (speedup 1.0000x reference)
import jax
import jax.numpy as jnp
from jax.experimental import pallas as pl


def kernel(x, edge_index, batch, Wl1, Wr1, b1, Wl2, Wr2, b2, W_out, b_out):
    raise NotImplementedError("write your pallas kernel here")


# speedup vs baseline: 2.6331x; 2.6331x over previous
import jax
import jax.numpy as jnp
from jax import lax
from jax.experimental import pallas as pl
from jax.experimental.pallas import tpu as pltpu
from jax.experimental.pallas import tpu_sc as plsc

_NC = 2
_NS = 16
_NW = _NC * _NS
_LANES = 128
_BLK = 256
_NPAD = 10240
_G = 64


def _tc_in_proj(xpad, Wl, Wr):
    nblk = xpad.shape[0] // _BLK
    H = Wl.shape[1]

    def body(x_ref, wl_ref, wr_ref, p_ref, xr_ref):
        xb = x_ref[...]
        p_ref[...] = jnp.dot(xb, wl_ref[...], preferred_element_type=jnp.float32)
        xr_ref[...] = jnp.dot(xb, wr_ref[...], preferred_element_type=jnp.float32)

    return pl.pallas_call(
        body,
        grid=(nblk,),
        in_specs=[
            pl.BlockSpec((_BLK, xpad.shape[1]), lambda i: (i, 0)),
            pl.BlockSpec(Wl.shape, lambda i: (0, 0)),
            pl.BlockSpec(Wr.shape, lambda i: (0, 0)),
        ],
        out_specs=[
            pl.BlockSpec((_BLK, H), lambda i: (i, 0)),
            pl.BlockSpec((_BLK, H), lambda i: (i, 0)),
        ],
        out_shape=[
            jax.ShapeDtypeStruct((xpad.shape[0], H), jnp.float32),
            jax.ShapeDtypeStruct((xpad.shape[0], H), jnp.float32),
        ],
    )(xpad, Wl, Wr)


def _sc_scatter_sum(P, srcR, dstR, z128):
    nchunk = srcR.shape[0] // _LANES
    rwk = nchunk // _NW
    rp = _NPAD // _NS
    mesh = plsc.VectorSubcoreMesh(core_axis_name="c", subcore_axis_name="s")

    def body(P_hbm, srcI, dstI, z128h, outS,
             accum, idx_s, idx_d, rows_v, sem):
        c = lax.axis_index("c")
        s = lax.axis_index("s")
        wid = c * _NS + s
        pltpu.sync_copy(z128h, rows_v)
        for t in range(rp // _LANES):
            pltpu.sync_copy(rows_v,
                            accum.at[pl.ds(s * rp + t * _LANES, _LANES)])
        plsc.subcore_barrier()

        def step(j, carry):
            base = (wid * rwk + j) * _LANES
            pltpu.sync_copy(srcI.at[pl.ds(base, _LANES)], idx_s)
            pltpu.sync_copy(dstI.at[pl.ds(base, _LANES)], idx_d)
            pltpu.async_copy(P_hbm.at[idx_s], rows_v, sem).wait()
            pltpu.sync_copy(rows_v, accum.at[idx_d], add=True)
            return carry

        lax.fori_loop(0, rwk, step, 0)
        plsc.subcore_barrier()
        for t in range(rp // _LANES):
            sl = pl.ds(s * rp + t * _LANES, _LANES)
            pltpu.sync_copy(accum.at[sl], rows_v)
            pltpu.sync_copy(rows_v, outS.at[c, sl])

    fn = pl.kernel(
        body,
        out_type=jax.ShapeDtypeStruct((_NC, _NPAD, 128), jnp.float32),
        mesh=mesh,
        scratch_types=[
            pltpu.VMEM_SHARED((_NPAD, 128), jnp.float32),
            pltpu.VMEM((_LANES,), jnp.int32),
            pltpu.VMEM((_LANES,), jnp.int32),
            pltpu.VMEM((_LANES, 128), jnp.float32),
            pltpu.SemaphoreType.DMA,
        ],
    )
    return fn(P, srcR, dstR, z128)


def _sc_degree(dstR, ones128, z128):
    nchunk = dstR.shape[0] // _LANES
    rwk = nchunk // _NW
    rp = _NPAD // _NS
    mesh = plsc.VectorSubcoreMesh(core_axis_name="c", subcore_axis_name="s")

    def body(dstI, onesh, z128h, outD, dacc, idx_d, ones_v, stage_v):
        c = lax.axis_index("c")
        s = lax.axis_index("s")
        wid = c * _NS + s
        pltpu.sync_copy(z128h, stage_v)
        for t in range(rp // _LANES):
            pltpu.sync_copy(stage_v,
                            dacc.at[pl.ds(s * rp + t * _LANES, _LANES)])
        pltpu.sync_copy(onesh, ones_v)
        plsc.subcore_barrier()

        def step(j, carry):
            base = (wid * rwk + j) * _LANES
            pltpu.sync_copy(dstI.at[pl.ds(base, _LANES)], idx_d)
            pltpu.sync_copy(ones_v, dacc.at[idx_d], add=True)
            return carry

        lax.fori_loop(0, rwk, step, 0)
        plsc.subcore_barrier()
        for t in range(rp // _LANES):
            sl = pl.ds(s * rp + t * _LANES, _LANES)
            pltpu.sync_copy(dacc.at[sl], stage_v)
            pltpu.sync_copy(stage_v, outD.at[c, sl])

    fn = pl.kernel(
        body,
        out_type=jax.ShapeDtypeStruct((_NC, _NPAD, 128), jnp.float32),
        mesh=mesh,
        scratch_types=[
            pltpu.VMEM_SHARED((_NPAD, 128), jnp.float32),
            pltpu.VMEM((_LANES,), jnp.int32),
            pltpu.VMEM((_LANES, 128), jnp.float32),
            pltpu.VMEM((_LANES, 128), jnp.float32),
        ],
    )
    return fn(dstR, ones128, z128)


def _tc_combine_mid(Sp, degp, XR, b1r, Wl2, Wr2):
    nblk = _NPAD // _BLK
    H = Wl2.shape[0]

    def body(s_ref, d_ref, xr_ref, b_ref, wl_ref, wr_ref, p_ref, xr2_ref):
        ssum = s_ref[0] + s_ref[1]
        deg = d_ref[0] + d_ref[1]
        inv = 1.0 / jnp.maximum(deg, 1.0)
        h = jnp.maximum(ssum * inv + b_ref[...] + xr_ref[...], 0.0)
        p_ref[...] = jnp.dot(h, wl_ref[...], preferred_element_type=jnp.float32)
        xr2_ref[...] = jnp.dot(h, wr_ref[...], preferred_element_type=jnp.float32)

    return pl.pallas_call(
        body,
        grid=(nblk,),
        in_specs=[
            pl.BlockSpec((2, _BLK, H), lambda i: (0, i, 0)),
            pl.BlockSpec((2, _BLK, H), lambda i: (0, i, 0)),
            pl.BlockSpec((_BLK, H), lambda i: (i, 0)),
            pl.BlockSpec((1, H), lambda i: (0, 0)),
            pl.BlockSpec((H, H), lambda i: (0, 0)),
            pl.BlockSpec((H, H), lambda i: (0, 0)),
        ],
        out_specs=[
            pl.BlockSpec((_BLK, H), lambda i: (i, 0)),
            pl.BlockSpec((_BLK, H), lambda i: (i, 0)),
        ],
        out_shape=[
            jax.ShapeDtypeStruct((_NPAD, H), jnp.float32),
            jax.ShapeDtypeStruct((_NPAD, H), jnp.float32),
        ],
    )(Sp, degp, XR, b1r, Wl2, Wr2)


def _tc_combine_pool_head(Sp, degp, XR, b2r, batchR, Wp, bp, C):
    nblk = _NPAD // _BLK
    H = XR.shape[1]

    def body(s_ref, d_ref, xr_ref, b_ref, bt_ref, wp_ref, bp_ref, out_ref, scr):
        i = pl.program_id(0)

        @pl.when(i == 0)
        def _init():
            scr[...] = jnp.full((_G, 128), -jnp.inf, dtype=jnp.float32)

        ssum = s_ref[0] + s_ref[1]
        deg = d_ref[0] + d_ref[1]
        inv = 1.0 / jnp.maximum(deg, 1.0)
        h = jnp.maximum(ssum * inv + b_ref[...] + xr_ref[...], 0.0)

        bt = bt_ref[...]
        g_lo = jnp.min(bt)
        g_hi = jnp.minimum(jnp.max(bt), _G - 1)
        row_g = lax.broadcasted_iota(jnp.int32, (_G, 128), 0)

        def upd(g, carry):
            m = bt == g
            v = jnp.max(jnp.where(m, h, -jnp.inf), axis=0, keepdims=True)
            upd_mat = jnp.where(row_g == g, v, -jnp.inf)
            scr[...] = jnp.maximum(scr[...], upd_mat)
            return carry

        lax.fori_loop(g_lo, g_hi + 1, upd, 0)

        @pl.when(i == nblk - 1)
        def _head():
            pooled = scr[...]
            logits = (jnp.dot(pooled, wp_ref[...],
                              preferred_element_type=jnp.float32)
                      + bp_ref[...])
            colmask = lax.broadcasted_iota(jnp.int32, (_G, 128), 1) < C
            lm = jnp.where(colmask, logits, -jnp.inf)
            mx = jnp.max(lm, axis=1, keepdims=True)
            se = jnp.sum(jnp.where(colmask, jnp.exp(lm - mx), 0.0),
                         axis=1, keepdims=True)
            out_ref[...] = lm - (jnp.log(se) + mx)

    return pl.pallas_call(
        body,
        grid=(nblk,),
        in_specs=[
            pl.BlockSpec((2, _BLK, H), lambda i: (0, i, 0)),
            pl.BlockSpec((2, _BLK, H), lambda i: (0, i, 0)),
            pl.BlockSpec((_BLK, H), lambda i: (i, 0)),
            pl.BlockSpec((1, H), lambda i: (0, 0)),
            pl.BlockSpec((_BLK, H), lambda i: (i, 0)),
            pl.BlockSpec((H, 128), lambda i: (0, 0)),
            pl.BlockSpec((1, 128), lambda i: (0, 0)),
        ],
        out_specs=pl.BlockSpec((_G, 128), lambda i: (0, 0)),
        out_shape=jax.ShapeDtypeStruct((_G, 128), jnp.float32),
        scratch_shapes=[pltpu.VMEM((_G, 128), jnp.float32)],
    )(Sp, degp, XR, b2r, batchR, Wp, bp)


def kernel(x, edge_index, batch, Wl1, Wr1, b1, Wl2, Wr2, b2, W_out, b_out):
    N, F = x.shape
    H = Wl1.shape[1]
    C = W_out.shape[1]
    E = edge_index.shape[1]

    xpad = jnp.pad(x, ((0, _NPAD - N), (0, 0)))
    epad = (-E) % (_NW * _LANES * 8)
    src = jnp.concatenate([edge_index[0], jnp.zeros((epad,), jnp.int32)])
    dst = jnp.concatenate([edge_index[1], jnp.full((epad,), N, jnp.int32)])
    srcR = src.astype(jnp.int32)
    dstR = dst.astype(jnp.int32)
    z128 = jnp.zeros((_LANES, 128), jnp.float32)
    ones128 = jnp.ones((_LANES, 128), jnp.float32)
    batchR = jnp.broadcast_to(
        jnp.pad(batch.astype(jnp.int32), (0, _NPAD - N),
                constant_values=_G)[:, None], (_NPAD, H))
    b1r = b1.reshape(1, H)
    b2r = b2.reshape(1, H)
    Wp = jnp.pad(W_out, ((0, 0), (0, 128 - C)))
    bp = jnp.pad(b_out, (0, 128 - C)).reshape(1, 128)

    degp = _sc_degree(dstR, ones128, z128)

    P1, XR1 = _tc_in_proj(xpad, Wl1, Wr1)
    S1 = _sc_scatter_sum(P1, srcR, dstR, z128)
    P2, XR2 = _tc_combine_mid(S1, degp, XR1, b1r, Wl2, Wr2)

    S2 = _sc_scatter_sum(P2, srcR, dstR, z128)
    outp = _tc_combine_pool_head(S2, degp, XR2, b2r, batchR, Wp, bp, C)
    return outp[:, :C]


# speedup vs baseline: 3.1343x; 1.1903x over previous
import jax
import jax.numpy as jnp
from jax import lax
from jax.experimental import pallas as pl
from jax.experimental.pallas import tpu as pltpu
from jax.experimental.pallas import tpu_sc as plsc

_NC = 2
_NS = 16
_NW = _NC * _NS
_LANES = 128
_BLK = 256
_NPAD = 10240
_G = 64


def _tc_in_proj(xpad, Wl, Wr):
    nblk = xpad.shape[0] // _BLK
    H = Wl.shape[1]

    def body(x_ref, wl_ref, wr_ref, p_ref, xr_ref):
        xb = x_ref[...]
        p_ref[...] = jnp.dot(xb, wl_ref[...], preferred_element_type=jnp.float32)
        xr_ref[...] = jnp.dot(xb, wr_ref[...], preferred_element_type=jnp.float32)

    return pl.pallas_call(
        body,
        grid=(nblk,),
        in_specs=[
            pl.BlockSpec((_BLK, xpad.shape[1]), lambda i: (i, 0)),
            pl.BlockSpec(Wl.shape, lambda i: (0, 0)),
            pl.BlockSpec(Wr.shape, lambda i: (0, 0)),
        ],
        out_specs=[
            pl.BlockSpec((_BLK, H), lambda i: (i, 0)),
            pl.BlockSpec((_BLK, H), lambda i: (i, 0)),
        ],
        out_shape=[
            jax.ShapeDtypeStruct((xpad.shape[0], H), jnp.float32),
            jax.ShapeDtypeStruct((xpad.shape[0], H), jnp.float32),
        ],
    )(xpad, Wl, Wr)


def _sc_scatter_sum(P, srcR, dstR, z128):
    nchunk = srcR.shape[0]
    rwk = nchunk // _NW
    rp = _NPAD // _NS
    mesh = plsc.VectorSubcoreMesh(core_axis_name="c", subcore_axis_name="s")

    grp = 8
    ngrp = rwk // grp

    def body(P_hbm, srcI, dstI, z128h, outS,
             accum, idx_s, idx_d, rows0, rows1, sem0, sem1):
        c = lax.axis_index("c")
        s = lax.axis_index("s")
        wid = c * _NS + s
        pltpu.sync_copy(z128h, rows0)
        for t in range(rp // _LANES):
            pltpu.sync_copy(rows0,
                            accum.at[pl.ds(s * rp + t * _LANES, _LANES)])
        plsc.subcore_barrier()

        bufs = ((rows0, sem0), (rows1, sem1))

        def group(g, carry):
            rowbase = wid * rwk + g * grp
            pltpu.sync_copy(srcI.at[pl.ds(rowbase, grp)], idx_s)
            pltpu.sync_copy(dstI.at[pl.ds(rowbase, grp)], idx_d)
            cp = pltpu.async_copy(P_hbm.at[idx_s.at[0]], rows0, sem0)
            for j in range(grp):
                rv, _ = bufs[j % 2]
                cp_next = None
                if j + 1 < grp:
                    rn, sn = bufs[(j + 1) % 2]
                    cp_next = pltpu.async_copy(P_hbm.at[idx_s.at[j + 1]], rn, sn)
                cp.wait()
                pltpu.sync_copy(rv, accum.at[idx_d.at[j]], add=True)
                cp = cp_next
            return carry

        lax.fori_loop(0, ngrp, group, 0)
        plsc.subcore_barrier()
        for t in range(rp // _LANES):
            sl = pl.ds(s * rp + t * _LANES, _LANES)
            pltpu.sync_copy(accum.at[sl], rows0)
            pltpu.sync_copy(rows0, outS.at[c, sl])

    fn = pl.kernel(
        body,
        out_type=jax.ShapeDtypeStruct((_NC, _NPAD, 128), jnp.float32),
        mesh=mesh,
        scratch_types=[
            pltpu.VMEM_SHARED((_NPAD, 128), jnp.float32),
            pltpu.VMEM((grp, _LANES), jnp.int32),
            pltpu.VMEM((grp, _LANES), jnp.int32),
            pltpu.VMEM((_LANES, 128), jnp.float32),
            pltpu.VMEM((_LANES, 128), jnp.float32),
            pltpu.SemaphoreType.DMA,
            pltpu.SemaphoreType.DMA,
        ],
    )
    return fn(P, srcR, dstR, z128)


def _sc_degree(dstR, ones128, z128):
    nchunk = dstR.shape[0]
    rwk = nchunk // _NW
    rp = _NPAD // _NS
    mesh = plsc.VectorSubcoreMesh(core_axis_name="c", subcore_axis_name="s")

    grp = 8
    ngrp = rwk // grp

    def body(dstI, onesh, z128h, outD, dacc, idx_d, ones_v):
        c = lax.axis_index("c")
        s = lax.axis_index("s")
        wid = c * _NS + s
        pltpu.sync_copy(z128h, ones_v)
        for t in range(rp // _LANES):
            pltpu.sync_copy(ones_v,
                            dacc.at[pl.ds(s * rp + t * _LANES, _LANES)])
        pltpu.sync_copy(onesh, ones_v)
        plsc.subcore_barrier()

        def group(g, carry):
            rowbase = wid * rwk + g * grp
            pltpu.sync_copy(dstI.at[pl.ds(rowbase, grp)], idx_d)
            for j in range(grp):
                pltpu.sync_copy(ones_v, dacc.at[idx_d.at[j]], add=True)
            return carry

        lax.fori_loop(0, ngrp, group, 0)
        plsc.subcore_barrier()
        for t in range(rp // _LANES):
            sl = pl.ds(s * rp + t * _LANES, _LANES)
            pltpu.sync_copy(dacc.at[sl], ones_v)
            pltpu.sync_copy(ones_v, outD.at[c, sl])

    fn = pl.kernel(
        body,
        out_type=jax.ShapeDtypeStruct((_NC, _NPAD, 128), jnp.float32),
        mesh=mesh,
        scratch_types=[
            pltpu.VMEM_SHARED((_NPAD, 128), jnp.float32),
            pltpu.VMEM((grp, _LANES), jnp.int32),
            pltpu.VMEM((_LANES, 128), jnp.float32),
        ],
    )
    return fn(dstR, ones128, z128)


def _tc_combine_mid(Sp, degp, XR, b1r, Wl2, Wr2):
    nblk = _NPAD // _BLK
    H = Wl2.shape[0]

    def body(s_ref, d_ref, xr_ref, b_ref, wl_ref, wr_ref, p_ref, xr2_ref):
        ssum = s_ref[0] + s_ref[1]
        deg = d_ref[0] + d_ref[1]
        inv = 1.0 / jnp.maximum(deg, 1.0)
        h = jnp.maximum(ssum * inv + b_ref[...] + xr_ref[...], 0.0)
        p_ref[...] = jnp.dot(h, wl_ref[...], preferred_element_type=jnp.float32)
        xr2_ref[...] = jnp.dot(h, wr_ref[...], preferred_element_type=jnp.float32)

    return pl.pallas_call(
        body,
        grid=(nblk,),
        in_specs=[
            pl.BlockSpec((2, _BLK, H), lambda i: (0, i, 0)),
            pl.BlockSpec((2, _BLK, H), lambda i: (0, i, 0)),
            pl.BlockSpec((_BLK, H), lambda i: (i, 0)),
            pl.BlockSpec((1, H), lambda i: (0, 0)),
            pl.BlockSpec((H, H), lambda i: (0, 0)),
            pl.BlockSpec((H, H), lambda i: (0, 0)),
        ],
        out_specs=[
            pl.BlockSpec((_BLK, H), lambda i: (i, 0)),
            pl.BlockSpec((_BLK, H), lambda i: (i, 0)),
        ],
        out_shape=[
            jax.ShapeDtypeStruct((_NPAD, H), jnp.float32),
            jax.ShapeDtypeStruct((_NPAD, H), jnp.float32),
        ],
    )(Sp, degp, XR, b1r, Wl2, Wr2)


def _tc_combine_pool_head(Sp, degp, XR, b2r, batchR, Wp, bp, C):
    nblk = _NPAD // _BLK
    H = XR.shape[1]

    def body(s_ref, d_ref, xr_ref, b_ref, bt_ref, wp_ref, bp_ref, out_ref, scr):
        i = pl.program_id(0)

        @pl.when(i == 0)
        def _init():
            scr[...] = jnp.full((_G, 128), -jnp.inf, dtype=jnp.float32)

        ssum = s_ref[0] + s_ref[1]
        deg = d_ref[0] + d_ref[1]
        inv = 1.0 / jnp.maximum(deg, 1.0)
        h = jnp.maximum(ssum * inv + b_ref[...] + xr_ref[...], 0.0)

        bt = bt_ref[...]
        g_lo = jnp.min(bt)
        g_hi = jnp.minimum(jnp.max(bt), _G - 1)
        row_g = lax.broadcasted_iota(jnp.int32, (_G, 128), 0)

        def upd(g, carry):
            m = bt == g
            v = jnp.max(jnp.where(m, h, -jnp.inf), axis=0, keepdims=True)
            upd_mat = jnp.where(row_g == g, v, -jnp.inf)
            scr[...] = jnp.maximum(scr[...], upd_mat)
            return carry

        lax.fori_loop(g_lo, g_hi + 1, upd, 0)

        @pl.when(i == nblk - 1)
        def _head():
            pooled = scr[...]
            logits = (jnp.dot(pooled, wp_ref[...],
                              preferred_element_type=jnp.float32)
                      + bp_ref[...])
            colmask = lax.broadcasted_iota(jnp.int32, (_G, 128), 1) < C
            lm = jnp.where(colmask, logits, -jnp.inf)
            mx = jnp.max(lm, axis=1, keepdims=True)
            se = jnp.sum(jnp.where(colmask, jnp.exp(lm - mx), 0.0),
                         axis=1, keepdims=True)
            out_ref[...] = lm - (jnp.log(se) + mx)

    return pl.pallas_call(
        body,
        grid=(nblk,),
        in_specs=[
            pl.BlockSpec((2, _BLK, H), lambda i: (0, i, 0)),
            pl.BlockSpec((2, _BLK, H), lambda i: (0, i, 0)),
            pl.BlockSpec((_BLK, H), lambda i: (i, 0)),
            pl.BlockSpec((1, H), lambda i: (0, 0)),
            pl.BlockSpec((_BLK, H), lambda i: (i, 0)),
            pl.BlockSpec((H, 128), lambda i: (0, 0)),
            pl.BlockSpec((1, 128), lambda i: (0, 0)),
        ],
        out_specs=pl.BlockSpec((_G, 128), lambda i: (0, 0)),
        out_shape=jax.ShapeDtypeStruct((_G, 128), jnp.float32),
        scratch_shapes=[pltpu.VMEM((_G, 128), jnp.float32)],
    )(Sp, degp, XR, b2r, batchR, Wp, bp)


def kernel(x, edge_index, batch, Wl1, Wr1, b1, Wl2, Wr2, b2, W_out, b_out):
    N, F = x.shape
    H = Wl1.shape[1]
    C = W_out.shape[1]
    E = edge_index.shape[1]

    xpad = jnp.pad(x, ((0, _NPAD - N), (0, 0)))
    epad = (-E) % (_NW * _LANES * 8)
    src = jnp.concatenate([edge_index[0], jnp.zeros((epad,), jnp.int32)])
    dst = jnp.concatenate([edge_index[1], jnp.full((epad,), N, jnp.int32)])
    srcR = src.astype(jnp.int32).reshape(-1, _LANES)
    dstR = dst.astype(jnp.int32).reshape(-1, _LANES)
    z128 = jnp.zeros((_LANES, 128), jnp.float32)
    ones128 = jnp.ones((_LANES, 128), jnp.float32)
    batchR = jnp.broadcast_to(
        jnp.pad(batch.astype(jnp.int32), (0, _NPAD - N),
                constant_values=_G)[:, None], (_NPAD, H))
    b1r = b1.reshape(1, H)
    b2r = b2.reshape(1, H)
    Wp = jnp.pad(W_out, ((0, 0), (0, 128 - C)))
    bp = jnp.pad(b_out, (0, 128 - C)).reshape(1, 128)

    degp = _sc_degree(dstR, ones128, z128)

    P1, XR1 = _tc_in_proj(xpad, Wl1, Wr1)
    S1 = _sc_scatter_sum(P1, srcR, dstR, z128)
    P2, XR2 = _tc_combine_mid(S1, degp, XR1, b1r, Wl2, Wr2)

    S2 = _sc_scatter_sum(P2, srcR, dstR, z128)
    outp = _tc_combine_pool_head(S2, degp, XR2, b2r, batchR, Wp, bp, C)
    return outp[:, :C]
